# Initial kernel scaffold; baseline (speedup 1.0000x reference)
import jax
import jax.numpy as jnp
from jax.experimental import pallas as pl


def kernel(atoms, edges):
    raise NotImplementedError("write your pallas kernel here")


# speedup vs baseline: 45.8661x; 45.8661x over previous
import functools

import jax
import jax.numpy as jnp
from jax import lax
from jax.experimental import pallas as pl
from jax.experimental.pallas import tpu as pltpu
from jax.experimental.pallas import tpu_sc as plsc

B, A, F, D = 64, 512, 128, 16
LANES = 16
NCHUNKS_F = F // LANES

NC, NS = 2, 16
NW = NC * NS
MOLS_PER_W = B // NW
ACHUNK = 128
NACH = A // ACHUNK


def _graph_pool_body(atoms_hbm, edges_hbm, out_hbm, atoms_v, edges_v, out_v, sem):
    wid = lax.axis_index("s") * NC + lax.axis_index("c")

    for m in range(MOLS_PER_W):
        b = wid * MOLS_PER_W + m
        pltpu.sync_copy(atoms_hbm.at[b], atoms_v)
        pltpu.sync_copy(edges_hbm.at[b], edges_v)

        for ch in range(NACH):
            def atom_body(a, carry, ch=ch):
                accs = [atoms_v[ch * ACHUNK + a, pl.ds(c * LANES, LANES)]
                        for c in range(NCHUNKS_F)]
                ev = edges_v[ch * ACHUNK + a, pl.ds(0, D)]
                for d in range(D):
                    row = ev[d]
                    for c in range(NCHUNKS_F):
                        accs[c] = jnp.maximum(
                            accs[c], atoms_v[row, pl.ds(c * LANES, LANES)])
                for c in range(NCHUNKS_F):
                    out_v[a, pl.ds(c * LANES, LANES)] = accs[c]
                return carry

            lax.fori_loop(0, ACHUNK, atom_body, 0)
            pltpu.sync_copy(out_v, out_hbm.at[b, pl.ds(ch * ACHUNK, ACHUNK)])


_graph_pool = pl.kernel(
    _graph_pool_body,
    out_type=jax.ShapeDtypeStruct((B, A, F), jnp.float32),
    mesh=plsc.VectorSubcoreMesh(core_axis_name="c", subcore_axis_name="s"),
    scratch_types=[
        pltpu.VMEM((A, F), jnp.float32),
        pltpu.VMEM((A, D), jnp.int32),
        pltpu.VMEM((ACHUNK, F), jnp.float32),
        pltpu.SemaphoreType.DMA,
    ],
    compiler_params=pltpu.CompilerParams(use_tc_tiling_on_sc=False),
)


def kernel(atoms, edges):
    return _graph_pool(atoms, edges.astype(jnp.int32))
